# Initial kernel scaffold; baseline (speedup 1.0000x reference)
import jax
import jax.numpy as jnp
from jax.experimental import pallas as pl


def kernel(query, tree_key, tree_value, random):
    raise NotImplementedError("write your pallas kernel here")


# speedup vs baseline: 3.7074x; 3.7074x over previous
import jax
import jax.numpy as jnp
from jax.experimental import pallas as pl

_DEPTH = 8
_H = 8
_LVL_OFF = (0, 8, 16, 24, 32, 48, 80, 144)
_LVL_PAD = (8, 8, 8, 8, 16, 32, 64, 128)
_KROWS = 272


def _tree_kernel(q_ref, key_ref, val_ref, sup_ref, val_out_ref):
    q = q_ref[...]
    TB, D = q.shape
    nq = -q
    sup_cols = []
    for h in range(_H):
        keys = key_ref[h]
        ix = jnp.zeros((TB, 1), jnp.int32)
        support = None
        for d in range(_DEPTH):
            n = 1 << d
            off = _LVL_OFF[d]
            kblk = keys[off:off + n, :]
            if n == 1:
                g = jnp.broadcast_to(kblk, (TB, 2 * D))
            else:
                iota = jax.lax.broadcasted_iota(jnp.int32, (TB, n), 1)
                oh = (iota == ix).astype(jnp.float32)
                g = jnp.dot(oh, kblk, preferred_element_type=jnp.float32)
            k1 = g[:, :D]
            k2 = g[:, D:]
            a1 = -jnp.logaddexp(-k1, nq)
            a2 = -jnp.logaddexp(-k2, q)
            s_el = jnp.logaddexp(a1, a2)
            s = -jax.nn.logsumexp(-s_el, axis=-1, keepdims=True)
            bit = s >= 0.0
            ix = 2 * ix + bit.astype(jnp.int32)
            s = jnp.where(bit, s, -s)
            support = s if support is None else -jnp.logaddexp(-support, -s)
        iota = jax.lax.broadcasted_iota(jnp.int32, (TB, 1 << _DEPTH), 1)
        oh = (iota == ix).astype(jnp.float32)
        val_out_ref[:, h, :] = jnp.dot(oh, val_ref[h],
                                       preferred_element_type=jnp.float32)
        sup_cols.append(support)
    S = jnp.concatenate(sup_cols, axis=1)
    sup_ref[...] = -jax.nn.logsumexp(-S, axis=-1, keepdims=True)


def kernel(query, tree_key, tree_value, random):
    B, D = query.shape
    H = tree_key.shape[0]
    out_dim = tree_value.shape[-1]
    TB = 256

    key2 = tree_key.reshape(H, (1 << _DEPTH) - 1, 2 * D)
    parts = []
    for d in range(_DEPTH):
        n = 1 << d
        blk = key2[:, n - 1:2 * n - 1, :]
        parts.append(jnp.pad(blk, ((0, 0), (0, _LVL_PAD[d] - n), (0, 0))))
    keyp = jnp.concatenate(parts, axis=1)

    sup, val = pl.pallas_call(
        _tree_kernel,
        grid=(B // TB,),
        in_specs=[
            pl.BlockSpec((TB, D), lambda b: (b, 0)),
            pl.BlockSpec((H, _KROWS, 2 * D), lambda b: (0, 0, 0)),
            pl.BlockSpec((H, 1 << _DEPTH, out_dim), lambda b: (0, 0, 0)),
        ],
        out_specs=[
            pl.BlockSpec((TB, 1), lambda b: (b, 0)),
            pl.BlockSpec((TB, H, out_dim), lambda b: (b, 0, 0)),
        ],
        out_shape=[
            jax.ShapeDtypeStruct((B, 1), jnp.float32),
            jax.ShapeDtypeStruct((B, H, out_dim), jnp.float32),
        ],
    )(query, keyp, tree_value)
    return sup.reshape(B), val.reshape(B * H, out_dim)


# speedup vs baseline: 12.8087x; 3.4549x over previous
import jax
import jax.numpy as jnp
from jax.experimental import pallas as pl

_DEPTH = 8
_H = 8
_LVL_OFF = (0, 8, 16, 24, 32, 48, 80, 144)
_LVL_PAD = (8, 8, 8, 8, 16, 32, 64, 128)
_KROWS = 272


def _exp_kernel(k_ref, ek_ref):
    ek_ref[...] = jnp.exp(-k_ref[...])


def _tree_kernel(q_ref, ek_ref, val_ref, sup_ref, val_out_ref):
    q = q_ref[...]
    TB, D = q.shape
    u = jnp.exp(-q)
    iu = jnp.exp(q)
    acc = jnp.zeros((TB, 1), jnp.float32)
    for h in range(_H):
        eks = ek_ref[h]
        ix = jnp.zeros((TB, 1), jnp.int32)
        for d in range(_DEPTH):
            n = 1 << d
            off = _LVL_OFF[d]
            ekblk = eks[off:off + n, :]
            if n == 1:
                g = jnp.broadcast_to(ekblk, (TB, 2 * D))
            else:
                iota = jax.lax.broadcasted_iota(jnp.int32, (TB, n), 1)
                oh = (iota == ix).astype(jnp.float32)
                g = jnp.dot(oh, ekblk, preferred_element_type=jnp.float32)
            a = g[:, :D] + u
            b = g[:, D:] + iu
            r = (a * b) / (a + b)
            t = jnp.sum(r, axis=-1, keepdims=True)
            bit = t <= 1.0
            ix = 2 * ix + bit.astype(jnp.int32)
            acc = acc + jnp.minimum(t, 1.0 / t)
        iota = jax.lax.broadcasted_iota(jnp.int32, (TB, 1 << _DEPTH), 1)
        oh = (iota == ix).astype(jnp.float32)
        val_out_ref[:, h, :] = jnp.dot(oh, val_ref[h],
                                       preferred_element_type=jnp.float32)
    sup_ref[...] = -jnp.log(acc)


def kernel(query, tree_key, tree_value, random):
    B, D = query.shape
    H = tree_key.shape[0]
    out_dim = tree_value.shape[-1]
    TB = 256

    key2 = tree_key.reshape(H, (1 << _DEPTH) - 1, 2 * D)
    parts = []
    for d in range(_DEPTH):
        n = 1 << d
        blk = key2[:, n - 1:2 * n - 1, :]
        parts.append(jnp.pad(blk, ((0, 0), (0, _LVL_PAD[d] - n), (0, 0))))
    keyp = jnp.concatenate(parts, axis=1)

    ek = pl.pallas_call(
        _exp_kernel,
        grid=(H,),
        in_specs=[pl.BlockSpec((1, _KROWS, 2 * D), lambda h: (h, 0, 0))],
        out_specs=pl.BlockSpec((1, _KROWS, 2 * D), lambda h: (h, 0, 0)),
        out_shape=jax.ShapeDtypeStruct((H, _KROWS, 2 * D), jnp.float32),
    )(keyp)

    sup, val = pl.pallas_call(
        _tree_kernel,
        grid=(B // TB,),
        in_specs=[
            pl.BlockSpec((TB, D), lambda b: (b, 0)),
            pl.BlockSpec((H, _KROWS, 2 * D), lambda b: (0, 0, 0)),
            pl.BlockSpec((H, 1 << _DEPTH, out_dim), lambda b: (0, 0, 0)),
        ],
        out_specs=[
            pl.BlockSpec((TB, 1), lambda b: (b, 0)),
            pl.BlockSpec((TB, H, out_dim), lambda b: (b, 0, 0)),
        ],
        out_shape=[
            jax.ShapeDtypeStruct((B, 1), jnp.float32),
            jax.ShapeDtypeStruct((B, H, out_dim), jnp.float32),
        ],
    )(query, ek, tree_value)
    return sup.reshape(B), val.reshape(B * H, out_dim)
